# Initial kernel scaffold; baseline (speedup 1.0000x reference)
import jax
import jax.numpy as jnp
from jax.experimental import pallas as pl


def kernel(x, edge_index, batch, conv0_W1, conv0_b1, conv0_W2, conv0_b2, conv1_W1, conv1_b1, conv1_W2, conv1_b2, conv2_W1, conv2_b1, conv2_W2, conv2_b2, conv3_W1, conv3_b1, conv3_W2, conv3_b2, conv4_W1, conv4_b1, conv4_W2, conv4_b2, fc1_W, fc1_b, fc2_W, fc2_b):
    raise NotImplementedError("write your pallas kernel here")


# speedup vs baseline: 10.4280x; 10.4280x over previous
import functools

import jax
import jax.numpy as jnp
from jax import lax
from jax.experimental import pallas as pl
from jax.experimental.pallas import tpu as pltpu
from jax.experimental.pallas import tpu_sc as plsc

N = 100000
E = 1600000
D = 32
HALF = 16
NG = 128
NCLS = 6
NSUB = 16
NCORE = 2

GRP = 80
GPC = 8
CHUNK = GRP * GPC
NCHUNK = E // CHUNK
WB = 1000
NWB = N // WB

NB = 2000
GRID = N // NB


def _sc_segsum_body(h2, src3, dst3, out, srcv, idxv, dstv, rows, stage, acc, sem):
    c = lax.axis_index("c")
    s = lax.axis_index("s")

    def _zrow(i, carry):
        stage[i] = jnp.zeros((HALF,), jnp.float32)
        return carry
    lax.fori_loop(0, WB, _zrow, 0)
    nblk = jnp.where(s < NWB % NSUB, NWB // NSUB + 1, NWB // NSUB)

    def _zblk(k, carry):
        b = s + k * NSUB
        pltpu.sync_copy(stage, acc.at[pl.ds(b * WB, WB)])
        return carry
    lax.fori_loop(0, nblk, _zblk, 0)
    plsc.subcore_barrier()

    nchunk = jnp.where(s < NCHUNK % NSUB, NCHUNK // NSUB + 1, NCHUNK // NSUB)

    def _chunk(t, carry):
        ci = s + t * NSUB
        pltpu.sync_copy(src3.at[ci], srcv)
        pltpu.sync_copy(dst3.at[ci], dstv)
        for j in range(GPC):
            for k in range(GRP // 16):
                v = srcv[j, pl.ds(k * 16, 16)]
                idxv[j, pl.ds(k * 16, 16)] = v * 2 + c
        cps = [
            pltpu.async_copy(h2.at[idxv.at[j]], rows.at[pl.ds(j * GRP, GRP)], sem)
            for j in range(GPC)
        ]
        for cp in cps:
            cp.wait()
        for j in range(GPC):
            pltpu.sync_copy(rows.at[pl.ds(j * GRP, GRP)], acc.at[dstv.at[j]], add=True)
        return carry
    lax.fori_loop(0, nchunk, _chunk, 0)
    plsc.subcore_barrier()

    def _wblk(k, carry):
        b = s + k * NSUB
        pltpu.sync_copy(acc.at[pl.ds(b * WB, WB)], stage)
        pltpu.sync_copy(stage, out.at[c, pl.ds(b * WB, WB)])
        return carry
    lax.fori_loop(0, nblk, _wblk, 0)


def _sc_segsum(h2, src3, dst3):
    return pl.kernel(
        _sc_segsum_body,
        out_type=jax.ShapeDtypeStruct((NCORE, N, HALF), jnp.float32),
        mesh=plsc.VectorSubcoreMesh(core_axis_name="c", subcore_axis_name="s"),
        compiler_params=pltpu.CompilerParams(use_tc_tiling_on_sc=False),
        scratch_types=[
            pltpu.VMEM((GPC, GRP), jnp.int32),
            pltpu.VMEM((GPC, GRP), jnp.int32),
            pltpu.VMEM((GPC, GRP), jnp.int32),
            pltpu.VMEM((CHUNK, HALF), jnp.float32),
            pltpu.VMEM((WB, HALF), jnp.float32),
            pltpu.VMEM_SHARED((N, HALF), jnp.float32),
            pltpu.SemaphoreType.DMA,
        ],
    )(h2, src3, dst3)


def _prep_body(x_ref, w_ref, out_ref):
    out_ref[...] = x_ref[...] * w_ref[...]


def _prep(x, w1):
    return pl.pallas_call(
        _prep_body,
        grid=(GRID,),
        in_specs=[
            pl.BlockSpec((NB, 1), lambda i: (i, 0)),
            pl.BlockSpec((1, D), lambda i: (0, 0)),
        ],
        out_specs=pl.BlockSpec((NB, D), lambda i: (i, 0)),
        out_shape=jax.ShapeDtypeStruct((N, D), jnp.float32),
    )(x, w1)


def _tc_layer_body(h_ref, agg_ref, b3_ref, w1_ref, b1_ref, w2_ref, b2_ref,
                   hout_ref, pool_ref):
    z = h_ref[...] + agg_ref[...]
    z = jnp.maximum(
        jnp.dot(z, w1_ref[...], preferred_element_type=jnp.float32) + b1_ref[...],
        0.0)
    z = jnp.dot(z, w2_ref[...], preferred_element_type=jnp.float32) + b2_ref[...]
    hn = jnp.maximum(z, 0.0)
    hout_ref[...] = hn
    b = b3_ref[0, 0, :]
    gids = lax.broadcasted_iota(jnp.int32, (NB, NG), 1)
    onehot = (b[:, None] == gids).astype(jnp.float32)
    pool = lax.dot_general(onehot, hn, (((0,), (0,)), ((), ())),
                           preferred_element_type=jnp.float32)

    @pl.when(pl.program_id(0) == 0)
    def _():
        pool_ref[...] = jnp.zeros_like(pool_ref)

    pool_ref[...] += pool


def _tc_layer(h, agg, batch3, w1, b1, w2, b2):
    return pl.pallas_call(
        _tc_layer_body,
        grid=(GRID,),
        in_specs=[
            pl.BlockSpec((NB, D), lambda i: (i, 0)),
            pl.BlockSpec((NB, D), lambda i: (i, 0)),
            pl.BlockSpec((1, 1, NB), lambda i: (i, 0, 0)),
            pl.BlockSpec((D, D), lambda i: (0, 0)),
            pl.BlockSpec((1, D), lambda i: (0, 0)),
            pl.BlockSpec((D, D), lambda i: (0, 0)),
            pl.BlockSpec((1, D), lambda i: (0, 0)),
        ],
        out_specs=[
            pl.BlockSpec((NB, D), lambda i: (i, 0)),
            pl.BlockSpec((NG, D), lambda i: (0, 0)),
        ],
        out_shape=[
            jax.ShapeDtypeStruct((N, D), jnp.float32),
            jax.ShapeDtypeStruct((NG, D), jnp.float32),
        ],
    )(h, agg, batch3, w1, b1, w2, b2)


def _head_body(g_ref, w1_ref, b1_ref, w2_ref, b2_ref, out_ref):
    g = jnp.maximum(
        jnp.dot(g_ref[...], w1_ref[...], preferred_element_type=jnp.float32)
        + b1_ref[...], 0.0)
    logits = jnp.dot(g, w2_ref[...], preferred_element_type=jnp.float32) + b2_ref[...]
    col = lax.broadcasted_iota(jnp.int32, (NG, 128), 1)
    valid = col < NCLS
    masked = jnp.where(valid, logits, -1e30)
    m = jnp.max(masked, axis=1, keepdims=True)
    e = jnp.where(valid, jnp.exp(logits - m), 0.0)
    lse = m + jnp.log(jnp.sum(e, axis=1, keepdims=True))
    out_ref[...] = logits - lse


def _head(g, w1, b1, w2, b2):
    return pl.pallas_call(
        _head_body,
        in_specs=[
            pl.BlockSpec((NG, 2 * 128), lambda: (0, 0)),
            pl.BlockSpec((2 * 128, D), lambda: (0, 0)),
            pl.BlockSpec((1, D), lambda: (0, 0)),
            pl.BlockSpec((D, 128), lambda: (0, 0)),
            pl.BlockSpec((1, 128), lambda: (0, 0)),
        ],
        out_specs=pl.BlockSpec((NG, 128), lambda: (0, 0)),
        out_shape=jax.ShapeDtypeStruct((NG, 128), jnp.float32),
    )(g, w1, b1, w2, b2)


def kernel(x, edge_index, batch,
           conv0_W1, conv0_b1, conv0_W2, conv0_b2,
           conv1_W1, conv1_b1, conv1_W2, conv1_b2,
           conv2_W1, conv2_b1, conv2_W2, conv2_b2,
           conv3_W1, conv3_b1, conv3_W2, conv3_b2,
           conv4_W1, conv4_b1, conv4_W2, conv4_b2,
           fc1_W, fc1_b, fc2_W, fc2_b):
    src3 = edge_index[0].reshape(NCHUNK, GPC, GRP)
    dst3 = edge_index[1].reshape(NCHUNK, GPC, GRP)
    batch3 = batch.reshape(GRID, 1, NB)

    eye = jnp.eye(D, dtype=jnp.float32)
    w1s = [eye, conv1_W1, conv2_W1, conv3_W1, conv4_W1]
    b1s = [conv0_b1, conv1_b1, conv2_b1, conv3_b1, conv4_b1]
    w2s = [conv0_W2, conv1_W2, conv2_W2, conv3_W2, conv4_W2]
    b2s = [conv0_b2, conv1_b2, conv2_b2, conv3_b2, conv4_b2]

    h = _prep(x, conv0_W1)
    pooled = []
    for i in range(5):
        h2 = h.reshape(2 * N, HALF)
        agg2 = _sc_segsum(h2, src3, dst3)
        agg = agg2.transpose(1, 0, 2).reshape(N, D)
        h, pool_i = _tc_layer(h, agg, batch3,
                              w1s[i], b1s[i].reshape(1, D),
                              w2s[i], b2s[i].reshape(1, D))
        pooled.append(pool_i)

    g = jnp.concatenate(pooled, axis=1)
    gp = jnp.pad(g, ((0, 0), (0, 2 * 128 - 5 * D)))
    fc1p = jnp.pad(fc1_W, ((0, 2 * 128 - 5 * D), (0, 0)))
    fc2p = jnp.pad(fc2_W, ((0, 0), (0, 128 - NCLS)))
    fc2bp = jnp.pad(fc2_b, (0, 128 - NCLS))
    out = _head(gp, fc1p, fc1_b.reshape(1, D), fc2p, fc2bp.reshape(1, 128))
    return out[:, :NCLS]


# speedup vs baseline: 11.1717x; 1.0713x over previous
import functools

import jax
import jax.numpy as jnp
from jax import lax
from jax.experimental import pallas as pl
from jax.experimental.pallas import tpu as pltpu
from jax.experimental.pallas import tpu_sc as plsc

N = 100000
E = 1600000
D = 32
HALF = 16
NG = 128
NCLS = 6
NSUB = 16
NCORE = 2

GRP = 80
GPC = 6
CHUNK = GRP * GPC
NCHUNK = 3360
EPAD = NCHUNK * CHUNK - E
CPS = NCHUNK // NSUB
WB = 500
NWB = N // WB

NB = 2000
GRID = N // NB


def _sc_segsum_body(h2, src3, dst3, out,
                    srcva, srcvb, idxva, idxvb, dstva, dstvb,
                    rowsa, rowsb, stage, acc, sema, semb):
    c = lax.axis_index("c")
    s = lax.axis_index("s")

    def _zrow(i, carry):
        stage[i] = jnp.zeros((HALF,), jnp.float32)
        return carry
    lax.fori_loop(0, WB, _zrow, 0)
    nblk = jnp.where(s < NWB % NSUB, NWB // NSUB + 1, NWB // NSUB)

    def _zblk(k, carry):
        b = s + k * NSUB
        pltpu.sync_copy(stage, acc.at[pl.ds(b * WB, WB)])
        return carry
    lax.fori_loop(0, nblk, _zblk, 0)
    plsc.subcore_barrier()

    def _fire(srcv, idxv, rows, sem, ci):
        pltpu.sync_copy(src3.at[ci], srcv)
        for j in range(GPC):
            for k in range(GRP // 16):
                v = srcv[j, pl.ds(k * 16, 16)]
                idxv[j, pl.ds(k * 16, 16)] = v * 2 + c
        for j in range(GPC):
            pltpu.async_copy(h2.at[idxv.at[j]], rows.at[pl.ds(j * GRP, GRP)], sem)

    def _drain(rows, sem):
        pltpu.make_async_copy(out.at[0, pl.ds(0, CHUNK)], rows, sem).wait()

    def _scatter(dstv, rows, ci):
        pltpu.sync_copy(dst3.at[ci], dstv)
        for j in range(GPC):
            pltpu.sync_copy(rows.at[pl.ds(j * GRP, GRP)], acc.at[dstv.at[j]], add=True)

    _fire(srcva, idxva, rowsa, sema, s)

    def _piter(k, carry):
        ci0 = s + (2 * k) * NSUB
        ci1 = s + (2 * k + 1) * NSUB
        _fire(srcvb, idxvb, rowsb, semb, ci1)
        _drain(rowsa, sema)
        _scatter(dstva, rowsa, ci0)

        @pl.when(k < CPS // 2 - 1)
        def _():
            _fire(srcva, idxva, rowsa, sema, s + (2 * k + 2) * NSUB)
        _drain(rowsb, semb)
        _scatter(dstvb, rowsb, ci1)
        return carry
    lax.fori_loop(0, CPS // 2, _piter, 0)
    plsc.subcore_barrier()

    def _wblk(k, carry):
        b = s + k * NSUB
        pltpu.sync_copy(acc.at[pl.ds(b * WB, WB)], stage)
        pltpu.sync_copy(stage, out.at[c, pl.ds(b * WB, WB)])
        return carry
    lax.fori_loop(0, nblk, _wblk, 0)


def _sc_segsum(h2, src3, dst3):
    return pl.kernel(
        _sc_segsum_body,
        out_type=jax.ShapeDtypeStruct((NCORE, N, HALF), jnp.float32),
        mesh=plsc.VectorSubcoreMesh(core_axis_name="c", subcore_axis_name="s"),
        compiler_params=pltpu.CompilerParams(use_tc_tiling_on_sc=False),
        scratch_types=[
            pltpu.VMEM((GPC, GRP), jnp.int32),
            pltpu.VMEM((GPC, GRP), jnp.int32),
            pltpu.VMEM((GPC, GRP), jnp.int32),
            pltpu.VMEM((GPC, GRP), jnp.int32),
            pltpu.VMEM((GPC, GRP), jnp.int32),
            pltpu.VMEM((GPC, GRP), jnp.int32),
            pltpu.VMEM((CHUNK, HALF), jnp.float32),
            pltpu.VMEM((CHUNK, HALF), jnp.float32),
            pltpu.VMEM((WB, HALF), jnp.float32),
            pltpu.VMEM_SHARED((N + 8, HALF), jnp.float32),
            pltpu.SemaphoreType.DMA,
            pltpu.SemaphoreType.DMA,
        ],
    )(h2, src3, dst3)


def _prep_body(x_ref, w_ref, out_ref):
    out_ref[...] = x_ref[...] * w_ref[...]


def _prep(x, w1):
    return pl.pallas_call(
        _prep_body,
        grid=(GRID,),
        in_specs=[
            pl.BlockSpec((NB, 1), lambda i: (i, 0)),
            pl.BlockSpec((1, D), lambda i: (0, 0)),
        ],
        out_specs=pl.BlockSpec((NB, D), lambda i: (i, 0)),
        out_shape=jax.ShapeDtypeStruct((N, D), jnp.float32),
    )(x, w1)


def _tc_layer_body(h_ref, agg_ref, b3_ref, w1_ref, b1_ref, w2_ref, b2_ref,
                   hout_ref, pool_ref):
    z = h_ref[...] + agg_ref[...]
    z = jnp.maximum(
        jnp.dot(z, w1_ref[...], preferred_element_type=jnp.float32) + b1_ref[...],
        0.0)
    z = jnp.dot(z, w2_ref[...], preferred_element_type=jnp.float32) + b2_ref[...]
    hn = jnp.maximum(z, 0.0)
    hout_ref[...] = hn
    b = b3_ref[0, 0, :]
    gids = lax.broadcasted_iota(jnp.int32, (NB, NG), 1)
    onehot = (b[:, None] == gids).astype(jnp.float32)
    pool = lax.dot_general(onehot, hn, (((0,), (0,)), ((), ())),
                           preferred_element_type=jnp.float32)

    @pl.when(pl.program_id(0) == 0)
    def _():
        pool_ref[...] = jnp.zeros_like(pool_ref)

    pool_ref[...] += pool


def _tc_layer(h, agg, batch3, w1, b1, w2, b2):
    return pl.pallas_call(
        _tc_layer_body,
        grid=(GRID,),
        in_specs=[
            pl.BlockSpec((NB, D), lambda i: (i, 0)),
            pl.BlockSpec((NB, D), lambda i: (i, 0)),
            pl.BlockSpec((1, 1, NB), lambda i: (i, 0, 0)),
            pl.BlockSpec((D, D), lambda i: (0, 0)),
            pl.BlockSpec((1, D), lambda i: (0, 0)),
            pl.BlockSpec((D, D), lambda i: (0, 0)),
            pl.BlockSpec((1, D), lambda i: (0, 0)),
        ],
        out_specs=[
            pl.BlockSpec((NB, D), lambda i: (i, 0)),
            pl.BlockSpec((NG, D), lambda i: (0, 0)),
        ],
        out_shape=[
            jax.ShapeDtypeStruct((N, D), jnp.float32),
            jax.ShapeDtypeStruct((NG, D), jnp.float32),
        ],
    )(h, agg, batch3, w1, b1, w2, b2)


def _head_body(g_ref, w1_ref, b1_ref, w2_ref, b2_ref, out_ref):
    g = jnp.maximum(
        jnp.dot(g_ref[...], w1_ref[...], preferred_element_type=jnp.float32)
        + b1_ref[...], 0.0)
    logits = jnp.dot(g, w2_ref[...], preferred_element_type=jnp.float32) + b2_ref[...]
    col = lax.broadcasted_iota(jnp.int32, (NG, 128), 1)
    valid = col < NCLS
    masked = jnp.where(valid, logits, -1e30)
    m = jnp.max(masked, axis=1, keepdims=True)
    e = jnp.where(valid, jnp.exp(logits - m), 0.0)
    lse = m + jnp.log(jnp.sum(e, axis=1, keepdims=True))
    out_ref[...] = logits - lse


def _head(g, w1, b1, w2, b2):
    return pl.pallas_call(
        _head_body,
        in_specs=[
            pl.BlockSpec((NG, 2 * 128), lambda: (0, 0)),
            pl.BlockSpec((2 * 128, D), lambda: (0, 0)),
            pl.BlockSpec((1, D), lambda: (0, 0)),
            pl.BlockSpec((D, 128), lambda: (0, 0)),
            pl.BlockSpec((1, 128), lambda: (0, 0)),
        ],
        out_specs=pl.BlockSpec((NG, 128), lambda: (0, 0)),
        out_shape=jax.ShapeDtypeStruct((NG, 128), jnp.float32),
    )(g, w1, b1, w2, b2)


def kernel(x, edge_index, batch,
           conv0_W1, conv0_b1, conv0_W2, conv0_b2,
           conv1_W1, conv1_b1, conv1_W2, conv1_b2,
           conv2_W1, conv2_b1, conv2_W2, conv2_b2,
           conv3_W1, conv3_b1, conv3_W2, conv3_b2,
           conv4_W1, conv4_b1, conv4_W2, conv4_b2,
           fc1_W, fc1_b, fc2_W, fc2_b):
    src_pad = jnp.concatenate(
        [edge_index[0], jnp.zeros((EPAD,), jnp.int32)])
    dst_pad = jnp.concatenate(
        [edge_index[1], N + (jnp.arange(EPAD, dtype=jnp.int32) % 8)])
    src3 = src_pad.reshape(NCHUNK, GPC, GRP)
    dst3 = dst_pad.reshape(NCHUNK, GPC, GRP)
    batch3 = batch.reshape(GRID, 1, NB)

    eye = jnp.eye(D, dtype=jnp.float32)
    w1s = [eye, conv1_W1, conv2_W1, conv3_W1, conv4_W1]
    b1s = [conv0_b1, conv1_b1, conv2_b1, conv3_b1, conv4_b1]
    w2s = [conv0_W2, conv1_W2, conv2_W2, conv3_W2, conv4_W2]
    b2s = [conv0_b2, conv1_b2, conv2_b2, conv3_b2, conv4_b2]

    h = _prep(x, conv0_W1)
    pooled = []
    for i in range(5):
        h2 = h.reshape(2 * N, HALF)
        agg2 = _sc_segsum(h2, src3, dst3)
        agg = agg2.transpose(1, 0, 2).reshape(N, D)
        h, pool_i = _tc_layer(h, agg, batch3,
                              w1s[i], b1s[i].reshape(1, D),
                              w2s[i], b2s[i].reshape(1, D))
        pooled.append(pool_i)

    g = jnp.concatenate(pooled, axis=1)
    gp = jnp.pad(g, ((0, 0), (0, 2 * 128 - 5 * D)))
    fc1p = jnp.pad(fc1_W, ((0, 2 * 128 - 5 * D), (0, 0)))
    fc2p = jnp.pad(fc2_W, ((0, 0), (0, 128 - NCLS)))
    fc2bp = jnp.pad(fc2_b, (0, 128 - NCLS))
    out = _head(gp, fc1p, fc1_b.reshape(1, D), fc2p, fc2bp.reshape(1, 128))
    return out[:, :NCLS]


# speedup vs baseline: 20.6237x; 1.8461x over previous
import jax
import jax.numpy as jnp
from jax import lax
from jax.experimental import pallas as pl
from jax.experimental.pallas import tpu as pltpu
from jax.experimental.pallas import tpu_sc as plsc

N = 100000
E = 1600000
D = 32
HALF = 16
NG = 128
NCLS = 6
NSUB = 16
NCORE = 2

GRP = 128
GPC = 3
CHUNK = GRP * GPC
NCHUNK = 4176
EPAD = NCHUNK * CHUNK - E
CPS = NCHUNK // NSUB
WB = 250
NWB = N // WB
OB = 128
NOB = N // OB

NROW = N // 4
NB = 1000
GRID = NROW // NB


def _sc_segsum_body(h2, src3, dst3, out,
                    srcv, idxv0, idxv1, idxv2, dstv0, dstv1, dstv2,
                    rows0, rows1, rows2, stage, widx, acc,
                    g0, g1, g2, sc0, sc1, sc2):
    c = lax.axis_index("c")
    s = lax.axis_index("s")
    idxvs = (idxv0, idxv1, idxv2)
    dstvs = (dstv0, dstv1, dstv2)
    rowss = (rows0, rows1, rows2)
    gsems = (g0, g1, g2)
    ssems = (sc0, sc1, sc2)

    def _zrow(i, carry):
        stage[i] = jnp.zeros((HALF,), jnp.float32)
        return carry
    lax.fori_loop(0, WB, _zrow, 0)

    def _zblk(k, carry):
        b = s + k * NSUB
        pltpu.sync_copy(stage, acc.at[pl.ds(b * WB, WB)])
        return carry
    lax.fori_loop(0, NWB // NSUB, _zblk, 0)

    @pl.when(s == 0)
    def _():
        pltpu.sync_copy(stage.at[pl.ds(0, 8)], acc.at[pl.ds(N, 8)])
    plsc.subcore_barrier()

    def _fire(b, ci):
        pltpu.sync_copy(src3.at[ci], srcv)
        for j in range(GPC):
            for k in range(GRP // 16):
                v = srcv[j, pl.ds(k * 16, 16)]
                idxvs[b][j, pl.ds(k * 16, 16)] = v * 2 + c
        for j in range(GPC):
            pltpu.async_copy(h2.at[idxvs[b].at[j]],
                             rowss[b].at[pl.ds(j * GRP, GRP)], gsems[b])

    def _gdrain(b):
        pltpu.make_async_copy(out.at[pl.ds(0, CHUNK)], rowss[b], gsems[b]).wait()

    def _scatter(b, ci):
        pltpu.sync_copy(dst3.at[ci], dstvs[b])
        for j in range(GPC):
            pltpu.async_copy(rowss[b].at[pl.ds(j * GRP, GRP)],
                             acc.at[dstvs[b].at[j]], ssems[b], add=True)

    def _sdrain(b):
        pltpu.make_async_copy(out.at[pl.ds(0, CHUNK)], rowss[b], ssems[b]).wait()

    _fire(0, s)
    _fire(1, s + NSUB)

    def _piter(k, carry):
        for b in range(3):
            t = 3 * k + b
            _gdrain(b)
            _scatter(b, s + t * NSUB)
            nb = (b + 2) % 3

            @pl.when(t + 2 < CPS)
            def _():
                @pl.when(t >= 1)
                def _():
                    _sdrain(nb)
                _fire(nb, s + (t + 2) * NSUB)
        return carry
    lax.fori_loop(0, CPS // 3, _piter, 0)
    _sdrain(0)
    _sdrain(1)
    _sdrain(2)
    plsc.subcore_barrier()

    iota2 = lax.iota(jnp.int32, 16) * 2
    nob = jnp.where(s < NOB % NSUB, NOB // NSUB + 1, NOB // NSUB)

    def _wblk(k, carry):
        b = s + k * NSUB
        off = b * OB
        pltpu.sync_copy(acc.at[pl.ds(off, OB)], stage.at[pl.ds(0, OB)])
        base = 2 * off + c
        for g in range(OB // 16):
            widx[0, pl.ds(g * 16, 16)] = base + 32 * g + iota2
        pltpu.sync_copy(stage.at[pl.ds(0, OB)], out.at[widx.at[0]])
        return carry
    lax.fori_loop(0, nob, _wblk, 0)

    @pl.when(s == NSUB - 1)
    def _():
        off = NOB * OB
        pltpu.sync_copy(acc.at[pl.ds(off, 32)], stage.at[pl.ds(0, 32)])
        base = 2 * off + c
        for g in range(2):
            widx[0, pl.ds(g * 16, 16)] = base + 32 * g + iota2
        pltpu.sync_copy(stage.at[pl.ds(0, 32)], out.at[widx.at[0, pl.ds(0, 32)]])


def _sc_segsum(h2, src3, dst3):
    return pl.kernel(
        _sc_segsum_body,
        out_type=jax.ShapeDtypeStruct((2 * N, HALF), jnp.float32),
        mesh=plsc.VectorSubcoreMesh(core_axis_name="c", subcore_axis_name="s"),
        compiler_params=pltpu.CompilerParams(use_tc_tiling_on_sc=False),
        scratch_types=[
            pltpu.VMEM((GPC, GRP), jnp.int32),
            pltpu.VMEM((GPC, GRP), jnp.int32),
            pltpu.VMEM((GPC, GRP), jnp.int32),
            pltpu.VMEM((GPC, GRP), jnp.int32),
            pltpu.VMEM((GPC, GRP), jnp.int32),
            pltpu.VMEM((GPC, GRP), jnp.int32),
            pltpu.VMEM((GPC, GRP), jnp.int32),
            pltpu.VMEM((CHUNK, HALF), jnp.float32),
            pltpu.VMEM((CHUNK, HALF), jnp.float32),
            pltpu.VMEM((CHUNK, HALF), jnp.float32),
            pltpu.VMEM((WB, HALF), jnp.float32),
            pltpu.VMEM((1, GRP), jnp.int32),
            pltpu.VMEM_SHARED((N + 8, HALF), jnp.float32),
            pltpu.SemaphoreType.DMA,
            pltpu.SemaphoreType.DMA,
            pltpu.SemaphoreType.DMA,
            pltpu.SemaphoreType.DMA,
            pltpu.SemaphoreType.DMA,
            pltpu.SemaphoreType.DMA,
        ],
    )(h2, src3, dst3)


def _prep_body(x4_ref, w4_ref, out_ref):
    out_ref[...] = jnp.dot(x4_ref[0], w4_ref[...],
                           preferred_element_type=jnp.float32)


def _prep(x4, w4):
    return pl.pallas_call(
        _prep_body,
        grid=(GRID,),
        in_specs=[
            pl.BlockSpec((1, NB, 4), lambda i: (i, 0, 0)),
            pl.BlockSpec((4, 128), lambda i: (0, 0)),
        ],
        out_specs=pl.BlockSpec((NB, 128), lambda i: (i, 0)),
        out_shape=jax.ShapeDtypeStruct((NROW, 128), jnp.float32),
    )(x4, w4)


def _tc_layer_body(h_ref, agg_ref, b0_ref, b1r_ref, b2r_ref, b3r_ref,
                   w1_ref, bias1_ref, w2_ref, bias2_ref,
                   hout_ref, pool_ref):
    z = h_ref[...] + agg_ref[...]
    z = jnp.maximum(
        jnp.dot(z, w1_ref[...], preferred_element_type=jnp.float32)
        + bias1_ref[...], 0.0)
    z = jnp.dot(z, w2_ref[...], preferred_element_type=jnp.float32) + bias2_ref[...]
    hn = jnp.maximum(z, 0.0)
    hout_ref[...] = hn

    gids = lax.broadcasted_iota(jnp.int32, (NB, NG), 1)
    pool = jnp.zeros((NG, D), jnp.float32)
    for i, bref in enumerate((b0_ref, b1r_ref, b2r_ref, b3r_ref)):
        bi = bref[0, 0, :]
        onehot = (bi[:, None] == gids).astype(jnp.float32)
        zi = hn[:, 32 * i:32 * (i + 1)]
        pool = pool + lax.dot_general(onehot, zi, (((0,), (0,)), ((), ())),
                                      preferred_element_type=jnp.float32)

    @pl.when(pl.program_id(0) == 0)
    def _():
        pool_ref[...] = jnp.zeros_like(pool_ref)

    pool_ref[...] += pool


def _tc_layer(h128, agg128, batches, w1d, b1d, w2d, b2d):
    return pl.pallas_call(
        _tc_layer_body,
        grid=(GRID,),
        in_specs=[
            pl.BlockSpec((NB, 128), lambda i: (i, 0)),
            pl.BlockSpec((NB, 128), lambda i: (i, 0)),
            pl.BlockSpec((1, 1, NB), lambda i: (i, 0, 0)),
            pl.BlockSpec((1, 1, NB), lambda i: (i, 0, 0)),
            pl.BlockSpec((1, 1, NB), lambda i: (i, 0, 0)),
            pl.BlockSpec((1, 1, NB), lambda i: (i, 0, 0)),
            pl.BlockSpec((128, 128), lambda i: (0, 0)),
            pl.BlockSpec((1, 128), lambda i: (0, 0)),
            pl.BlockSpec((128, 128), lambda i: (0, 0)),
            pl.BlockSpec((1, 128), lambda i: (0, 0)),
        ],
        out_specs=[
            pl.BlockSpec((NB, 128), lambda i: (i, 0)),
            pl.BlockSpec((NG, D), lambda i: (0, 0)),
        ],
        out_shape=[
            jax.ShapeDtypeStruct((NROW, 128), jnp.float32),
            jax.ShapeDtypeStruct((NG, D), jnp.float32),
        ],
    )(h128, agg128, batches[0], batches[1], batches[2], batches[3],
      w1d, b1d, w2d, b2d)


def _head_body(g_ref, w1_ref, b1_ref, w2_ref, b2_ref, out_ref):
    g = jnp.maximum(
        jnp.dot(g_ref[...], w1_ref[...], preferred_element_type=jnp.float32)
        + b1_ref[...], 0.0)
    logits = jnp.dot(g, w2_ref[...], preferred_element_type=jnp.float32) + b2_ref[...]
    col = lax.broadcasted_iota(jnp.int32, (NG, 128), 1)
    valid = col < NCLS
    masked = jnp.where(valid, logits, -1e30)
    m = jnp.max(masked, axis=1, keepdims=True)
    e = jnp.where(valid, jnp.exp(logits - m), 0.0)
    lse = m + jnp.log(jnp.sum(e, axis=1, keepdims=True))
    out_ref[...] = logits - lse


def _head(g, w1, b1, w2, b2):
    return pl.pallas_call(
        _head_body,
        in_specs=[
            pl.BlockSpec((NG, 2 * 128), lambda: (0, 0)),
            pl.BlockSpec((2 * 128, D), lambda: (0, 0)),
            pl.BlockSpec((1, D), lambda: (0, 0)),
            pl.BlockSpec((D, 128), lambda: (0, 0)),
            pl.BlockSpec((1, 128), lambda: (0, 0)),
        ],
        out_specs=pl.BlockSpec((NG, 128), lambda: (0, 0)),
        out_shape=jax.ShapeDtypeStruct((NG, 128), jnp.float32),
    )(g, w1, b1, w2, b2)


def kernel(x, edge_index, batch,
           conv0_W1, conv0_b1, conv0_W2, conv0_b2,
           conv1_W1, conv1_b1, conv1_W2, conv1_b2,
           conv2_W1, conv2_b1, conv2_W2, conv2_b2,
           conv3_W1, conv3_b1, conv3_W2, conv3_b2,
           conv4_W1, conv4_b1, conv4_W2, conv4_b2,
           fc1_W, fc1_b, fc2_W, fc2_b):
    src_pad = jnp.concatenate(
        [edge_index[0], jnp.zeros((EPAD,), jnp.int32)])
    dst_pad = jnp.concatenate(
        [edge_index[1], N + (jnp.arange(EPAD, dtype=jnp.int32) % 8)])
    src3 = src_pad.reshape(NCHUNK, GPC, GRP)
    dst3 = dst_pad.reshape(NCHUNK, GPC, GRP)

    b4 = batch.reshape(NROW, 4)
    batches = [b4[:, i].reshape(GRID, 1, NB) for i in range(4)]

    eye = jnp.eye(D, dtype=jnp.float32)
    i4 = jnp.eye(4, dtype=jnp.float32)
    w1s = [eye, conv1_W1, conv2_W1, conv3_W1, conv4_W1]
    b1s = [conv0_b1, conv1_b1, conv2_b1, conv3_b1, conv4_b1]
    w2s = [conv0_W2, conv1_W2, conv2_W2, conv3_W2, conv4_W2]
    b2s = [conv0_b2, conv1_b2, conv2_b2, conv3_b2, conv4_b2]

    w1d = [jnp.kron(i4, w) for w in w1s]
    w2d = [jnp.kron(i4, w) for w in w2s]
    b1d = [jnp.tile(b, 4).reshape(1, 128) for b in b1s]
    b2d = [jnp.tile(b, 4).reshape(1, 128) for b in b2s]

    x4 = x.reshape(GRID, NB, 4)
    w4 = jnp.kron(i4, conv0_W1)
    h128 = _prep(x4, w4)

    pooled = []
    for i in range(5):
        h2 = h128.reshape(2 * N, HALF)
        agg2 = _sc_segsum(h2, src3, dst3)
        agg128 = agg2.reshape(NROW, 128)
        h128, pool_i = _tc_layer(h128, agg128, batches,
                                 w1d[i], b1d[i], w2d[i], b2d[i])
        pooled.append(pool_i)

    g = jnp.concatenate(pooled, axis=1)
    gp = jnp.pad(g, ((0, 0), (0, 2 * 128 - 5 * D)))
    fc1p = jnp.pad(fc1_W, ((0, 2 * 128 - 5 * D), (0, 0)))
    fc2p = jnp.pad(fc2_W, ((0, 0), (0, 128 - NCLS)))
    fc2bp = jnp.pad(fc2_b, (0, 128 - NCLS))
    out = _head(gp, fc1p, fc1_b.reshape(1, D), fc2p, fc2bp.reshape(1, 128))
    return out[:, :NCLS]


# speedup vs baseline: 27.1685x; 1.3173x over previous
import jax
import jax.numpy as jnp
from jax import lax
from jax.experimental import pallas as pl
from jax.experimental.pallas import tpu as pltpu
from jax.experimental.pallas import tpu_sc as plsc

N = 100000
E = 1600000
D = 32
HALF = 16
NG = 128
NCLS = 6
NSUB = 16
NCORE = 2

GRP = 128
GPC = 3
CHUNK = GRP * GPC
NCHUNK = 4176
EPAD = NCHUNK * CHUNK - E
CPS = NCHUNK // NSUB
WB = 250
NWB = N // WB
OB = 128
NOB = N // OB

NROW = N // 4
NB = 1000
GRID = NROW // NB


def _sc_segsum_body(h2, e4, out,
                    ebuf0, ebuf1, ebuf2, idxv0, idxv1, idxv2,
                    dstv0, dstv1, dstv2,
                    rows0, rows1, rows2, stage, widx, acc,
                    g0, g1, g2, sc0, sc1, sc2, e0, e1, e2):
    c = lax.axis_index("c")
    s = lax.axis_index("s")
    ebufs = (ebuf0, ebuf1, ebuf2)
    idxvs = (idxv0, idxv1, idxv2)
    dstvs = (dstv0, dstv1, dstv2)
    rowss = (rows0, rows1, rows2)
    gsems = (g0, g1, g2)
    ssems = (sc0, sc1, sc2)
    esems = (e0, e1, e2)

    def _zrow(i, carry):
        stage[i] = jnp.zeros((HALF,), jnp.float32)
        return carry
    lax.fori_loop(0, WB, _zrow, 0)

    def _zblk(k, carry):
        b = s + k * NSUB
        pltpu.sync_copy(stage, acc.at[pl.ds(b * WB, WB)])
        return carry
    lax.fori_loop(0, NWB // NSUB, _zblk, 0)

    @pl.when(s == 0)
    def _():
        pltpu.sync_copy(stage.at[pl.ds(0, 8)], acc.at[pl.ds(N, 8)])
    plsc.subcore_barrier()

    def _eprefetch(b, ci):
        pltpu.async_copy(e4.at[ci], ebufs[b], esems[b])

    def _ewait(b):
        pltpu.make_async_copy(e4.at[0], ebufs[b], esems[b]).wait()

    def _fire(b):
        for j in range(GPC):
            for k in range(GRP // 16):
                v = ebufs[b][0, j, pl.ds(k * 16, 16)]
                idxvs[b][j, pl.ds(k * 16, 16)] = v * 2 + c
                dstvs[b][j, pl.ds(k * 16, 16)] = ebufs[b][1, j, pl.ds(k * 16, 16)]
        for j in range(GPC):
            pltpu.async_copy(h2.at[idxvs[b].at[j]],
                             rowss[b].at[pl.ds(j * GRP, GRP)], gsems[b])

    def _gdrain(b):
        pltpu.make_async_copy(out.at[pl.ds(0, CHUNK)], rowss[b], gsems[b]).wait()

    def _scatter(b):
        for j in range(GPC):
            pltpu.async_copy(rowss[b].at[pl.ds(j * GRP, GRP)],
                             acc.at[dstvs[b].at[j]], ssems[b], add=True)

    def _sdrain(b):
        pltpu.make_async_copy(out.at[pl.ds(0, CHUNK)], rowss[b], ssems[b]).wait()

    _eprefetch(0, s)
    _eprefetch(1, s + NSUB)
    _eprefetch(2, s + 2 * NSUB)
    _ewait(0)
    _fire(0)
    _ewait(1)
    _fire(1)

    def _piter(k, carry):
        for b in range(3):
            t = 3 * k + b
            _gdrain(b)
            _scatter(b)
            nb = (b + 2) % 3

            @pl.when(t + 2 < CPS)
            def _():
                @pl.when(t >= 1)
                def _():
                    _sdrain(nb)
                _ewait(nb)
                _fire(nb)

            @pl.when(t + 3 < CPS)
            def _():
                _eprefetch(b, s + (t + 3) * NSUB)
        return carry
    lax.fori_loop(0, CPS // 3, _piter, 0)
    _sdrain(0)
    _sdrain(1)
    _sdrain(2)
    plsc.subcore_barrier()

    iota2 = lax.iota(jnp.int32, 16) * 2
    nob = jnp.where(s < NOB % NSUB, NOB // NSUB + 1, NOB // NSUB)

    def _wblk(k, carry):
        b = s + k * NSUB
        off = b * OB
        pltpu.sync_copy(acc.at[pl.ds(off, OB)], stage.at[pl.ds(0, OB)])
        base = 2 * off + c
        for g in range(OB // 16):
            widx[0, pl.ds(g * 16, 16)] = base + 32 * g + iota2
        pltpu.sync_copy(stage.at[pl.ds(0, OB)], out.at[widx.at[0]])
        return carry
    lax.fori_loop(0, nob, _wblk, 0)

    @pl.when(s == NSUB - 1)
    def _():
        off = NOB * OB
        pltpu.sync_copy(acc.at[pl.ds(off, 32)], stage.at[pl.ds(0, 32)])
        base = 2 * off + c
        for g in range(2):
            widx[0, pl.ds(g * 16, 16)] = base + 32 * g + iota2
        pltpu.sync_copy(stage.at[pl.ds(0, 32)], out.at[widx.at[0, pl.ds(0, 32)]])


def _sc_segsum(h2, e4):
    return pl.kernel(
        _sc_segsum_body,
        out_type=jax.ShapeDtypeStruct((2 * N, HALF), jnp.float32),
        mesh=plsc.VectorSubcoreMesh(core_axis_name="c", subcore_axis_name="s"),
        compiler_params=pltpu.CompilerParams(use_tc_tiling_on_sc=False),
        scratch_types=[
            pltpu.VMEM((2, GPC, GRP), jnp.int32),
            pltpu.VMEM((2, GPC, GRP), jnp.int32),
            pltpu.VMEM((2, GPC, GRP), jnp.int32),
            pltpu.VMEM((GPC, GRP), jnp.int32),
            pltpu.VMEM((GPC, GRP), jnp.int32),
            pltpu.VMEM((GPC, GRP), jnp.int32),
            pltpu.VMEM((GPC, GRP), jnp.int32),
            pltpu.VMEM((GPC, GRP), jnp.int32),
            pltpu.VMEM((GPC, GRP), jnp.int32),
            pltpu.VMEM((CHUNK, HALF), jnp.float32),
            pltpu.VMEM((CHUNK, HALF), jnp.float32),
            pltpu.VMEM((CHUNK, HALF), jnp.float32),
            pltpu.VMEM((WB, HALF), jnp.float32),
            pltpu.VMEM((1, GRP), jnp.int32),
            pltpu.VMEM_SHARED((N + 8, HALF), jnp.float32),
            pltpu.SemaphoreType.DMA,
            pltpu.SemaphoreType.DMA,
            pltpu.SemaphoreType.DMA,
            pltpu.SemaphoreType.DMA,
            pltpu.SemaphoreType.DMA,
            pltpu.SemaphoreType.DMA,
            pltpu.SemaphoreType.DMA,
            pltpu.SemaphoreType.DMA,
            pltpu.SemaphoreType.DMA,
        ],
    )(h2, e4)


def _prep_body(x4_ref, w4_ref, out_ref):
    out_ref[...] = jnp.dot(x4_ref[0], w4_ref[...],
                           preferred_element_type=jnp.float32)


def _prep(x4, w4):
    return pl.pallas_call(
        _prep_body,
        grid=(GRID,),
        in_specs=[
            pl.BlockSpec((1, NB, 4), lambda i: (i, 0, 0)),
            pl.BlockSpec((4, 128), lambda i: (0, 0)),
        ],
        out_specs=pl.BlockSpec((NB, 128), lambda i: (i, 0)),
        out_shape=jax.ShapeDtypeStruct((NROW, 128), jnp.float32),
    )(x4, w4)


def _tc_layer_body(h_ref, agg_ref, b0_ref, b1r_ref, b2r_ref, b3r_ref,
                   w1_ref, bias1_ref, w2_ref, bias2_ref,
                   hout_ref, pool_ref):
    z = h_ref[...] + agg_ref[...]
    z = jnp.maximum(
        jnp.dot(z, w1_ref[...], preferred_element_type=jnp.float32)
        + bias1_ref[...], 0.0)
    z = jnp.dot(z, w2_ref[...], preferred_element_type=jnp.float32) + bias2_ref[...]
    hn = jnp.maximum(z, 0.0)
    hout_ref[...] = hn

    gids = lax.broadcasted_iota(jnp.int32, (NB, NG), 1)
    pool = jnp.zeros((NG, D), jnp.float32)
    for i, bref in enumerate((b0_ref, b1r_ref, b2r_ref, b3r_ref)):
        bi = bref[0, 0, :]
        onehot = (bi[:, None] == gids).astype(jnp.float32)
        zi = hn[:, 32 * i:32 * (i + 1)]
        pool = pool + lax.dot_general(onehot, zi, (((0,), (0,)), ((), ())),
                                      preferred_element_type=jnp.float32)

    @pl.when(pl.program_id(0) == 0)
    def _():
        pool_ref[...] = jnp.zeros_like(pool_ref)

    pool_ref[...] += pool


def _tc_layer(h128, agg128, batches, w1d, b1d, w2d, b2d):
    return pl.pallas_call(
        _tc_layer_body,
        grid=(GRID,),
        in_specs=[
            pl.BlockSpec((NB, 128), lambda i: (i, 0)),
            pl.BlockSpec((NB, 128), lambda i: (i, 0)),
            pl.BlockSpec((1, 1, NB), lambda i: (i, 0, 0)),
            pl.BlockSpec((1, 1, NB), lambda i: (i, 0, 0)),
            pl.BlockSpec((1, 1, NB), lambda i: (i, 0, 0)),
            pl.BlockSpec((1, 1, NB), lambda i: (i, 0, 0)),
            pl.BlockSpec((128, 128), lambda i: (0, 0)),
            pl.BlockSpec((1, 128), lambda i: (0, 0)),
            pl.BlockSpec((128, 128), lambda i: (0, 0)),
            pl.BlockSpec((1, 128), lambda i: (0, 0)),
        ],
        out_specs=[
            pl.BlockSpec((NB, 128), lambda i: (i, 0)),
            pl.BlockSpec((NG, D), lambda i: (0, 0)),
        ],
        out_shape=[
            jax.ShapeDtypeStruct((NROW, 128), jnp.float32),
            jax.ShapeDtypeStruct((NG, D), jnp.float32),
        ],
    )(h128, agg128, batches[0], batches[1], batches[2], batches[3],
      w1d, b1d, w2d, b2d)


def _head_body(g_ref, w1_ref, b1_ref, w2_ref, b2_ref, out_ref):
    g = jnp.maximum(
        jnp.dot(g_ref[...], w1_ref[...], preferred_element_type=jnp.float32)
        + b1_ref[...], 0.0)
    logits = jnp.dot(g, w2_ref[...], preferred_element_type=jnp.float32) + b2_ref[...]
    col = lax.broadcasted_iota(jnp.int32, (NG, 128), 1)
    valid = col < NCLS
    masked = jnp.where(valid, logits, -1e30)
    m = jnp.max(masked, axis=1, keepdims=True)
    e = jnp.where(valid, jnp.exp(logits - m), 0.0)
    lse = m + jnp.log(jnp.sum(e, axis=1, keepdims=True))
    out_ref[...] = logits - lse


def _head(g, w1, b1, w2, b2):
    return pl.pallas_call(
        _head_body,
        in_specs=[
            pl.BlockSpec((NG, 2 * 128), lambda: (0, 0)),
            pl.BlockSpec((2 * 128, D), lambda: (0, 0)),
            pl.BlockSpec((1, D), lambda: (0, 0)),
            pl.BlockSpec((D, 128), lambda: (0, 0)),
            pl.BlockSpec((1, 128), lambda: (0, 0)),
        ],
        out_specs=pl.BlockSpec((NG, 128), lambda: (0, 0)),
        out_shape=jax.ShapeDtypeStruct((NG, 128), jnp.float32),
    )(g, w1, b1, w2, b2)


def kernel(x, edge_index, batch,
           conv0_W1, conv0_b1, conv0_W2, conv0_b2,
           conv1_W1, conv1_b1, conv1_W2, conv1_b2,
           conv2_W1, conv2_b1, conv2_W2, conv2_b2,
           conv3_W1, conv3_b1, conv3_W2, conv3_b2,
           conv4_W1, conv4_b1, conv4_W2, conv4_b2,
           fc1_W, fc1_b, fc2_W, fc2_b):
    src_pad = jnp.concatenate(
        [edge_index[0], jnp.zeros((EPAD,), jnp.int32)])
    dst_pad = jnp.concatenate(
        [edge_index[1], N + (jnp.arange(EPAD, dtype=jnp.int32) % 8)])
    e4 = jnp.stack([src_pad.reshape(NCHUNK, GPC, GRP),
                    dst_pad.reshape(NCHUNK, GPC, GRP)], axis=1)

    bt = batch.reshape(NROW, 4).T
    batches = [bt[i].reshape(GRID, 1, NB) for i in range(4)]

    eye = jnp.eye(D, dtype=jnp.float32)
    i4 = jnp.eye(4, dtype=jnp.float32)
    w1s = [eye, conv1_W1, conv2_W1, conv3_W1, conv4_W1]
    b1s = [conv0_b1, conv1_b1, conv2_b1, conv3_b1, conv4_b1]
    w2s = [conv0_W2, conv1_W2, conv2_W2, conv3_W2, conv4_W2]
    b2s = [conv0_b2, conv1_b2, conv2_b2, conv3_b2, conv4_b2]

    w1d = [jnp.kron(i4, w) for w in w1s]
    w2d = [jnp.kron(i4, w) for w in w2s]
    b1d = [jnp.tile(b, 4).reshape(1, 128) for b in b1s]
    b2d = [jnp.tile(b, 4).reshape(1, 128) for b in b2s]

    x4 = x.reshape(GRID, NB, 4)
    w4 = jnp.kron(i4, conv0_W1)
    h128 = _prep(x4, w4)

    pooled = []
    for i in range(5):
        h2 = h128.reshape(2 * N, HALF)
        agg2 = _sc_segsum(h2, e4)
        agg128 = agg2.reshape(NROW, 128)
        h128, pool_i = _tc_layer(h128, agg128, batches,
                                 w1d[i], b1d[i], w2d[i], b2d[i])
        pooled.append(pool_i)

    g = jnp.concatenate(pooled, axis=1)
    gp = jnp.pad(g, ((0, 0), (0, 2 * 128 - 5 * D)))
    fc1p = jnp.pad(fc1_W, ((0, 2 * 128 - 5 * D), (0, 0)))
    fc2p = jnp.pad(fc2_W, ((0, 0), (0, 128 - NCLS)))
    fc2bp = jnp.pad(fc2_b, (0, 128 - NCLS))
    out = _head(gp, fc1p, fc1_b.reshape(1, D), fc2p, fc2bp.reshape(1, 128))
    return out[:, :NCLS]


# speedup vs baseline: 27.5922x; 1.0156x over previous
import jax
import jax.numpy as jnp
from jax import lax
from jax.experimental import pallas as pl
from jax.experimental.pallas import tpu as pltpu
from jax.experimental.pallas import tpu_sc as plsc

N = 100000
E = 1600000
D = 32
HALF = 16
NG = 128
NCLS = 6
NSUB = 16
NCORE = 2

GRP = 128
GPC = 3
CHUNK = GRP * GPC
NCHUNK = 4176
EPAD = NCHUNK * CHUNK - E
CPS = NCHUNK // NSUB
WB = 250
NWB = N // WB
OB = 128
NOB = N // OB

NROW = N // 4
NB = 1000
GRID = NROW // NB


def _sc_segsum_body(h2, e4, out,
                    ebuf0, ebuf1, ebuf2, idxv0, idxv1, idxv2,
                    dstv0, dstv1, dstv2,
                    rows0, rows1, rows2, stage, widx, acc,
                    g0, g1, g2, sc0, sc1, sc2, e0, e1, e2):
    c = lax.axis_index("c")
    s = lax.axis_index("s")
    ebufs = (ebuf0, ebuf1, ebuf2)
    idxvs = (idxv0, idxv1, idxv2)
    dstvs = (dstv0, dstv1, dstv2)
    rowss = (rows0, rows1, rows2)
    gsems = (g0, g1, g2)
    ssems = (sc0, sc1, sc2)
    esems = (e0, e1, e2)

    def _zrow(i, carry):
        stage[i] = jnp.zeros((HALF,), jnp.float32)
        return carry
    lax.fori_loop(0, WB, _zrow, 0)

    def _zblk(k, carry):
        b = s + k * NSUB
        pltpu.sync_copy(stage, acc.at[pl.ds(b * WB, WB)])
        return carry
    lax.fori_loop(0, NWB // NSUB, _zblk, 0)

    @pl.when(s == 0)
    def _():
        pltpu.sync_copy(stage.at[pl.ds(0, 8)], acc.at[pl.ds(N, 8)])
    plsc.subcore_barrier()

    def _eprefetch(b, ci):
        pltpu.async_copy(e4.at[ci], ebufs[b], esems[b])

    def _ewait(b):
        pltpu.make_async_copy(e4.at[0], ebufs[b], esems[b]).wait()

    def _fire(b):
        for j in range(GPC):
            for k in range(GRP // 16):
                v = ebufs[b][0, j, pl.ds(k * 16, 16)]
                idxvs[b][j, pl.ds(k * 16, 16)] = v * 2 + c
                dstvs[b][j, pl.ds(k * 16, 16)] = ebufs[b][1, j, pl.ds(k * 16, 16)]
        for j in range(GPC):
            pltpu.async_copy(h2.at[idxvs[b].at[j]],
                             rowss[b].at[pl.ds(j * GRP, GRP)], gsems[b])

    def _gdrain(b):
        pltpu.make_async_copy(out.at[pl.ds(0, CHUNK)], rowss[b], gsems[b]).wait()

    def _scatter(b):
        for j in range(GPC):
            pltpu.async_copy(rowss[b].at[pl.ds(j * GRP, GRP)],
                             acc.at[dstvs[b].at[j]], ssems[b], add=True)

    def _sdrain(b):
        pltpu.make_async_copy(out.at[pl.ds(0, CHUNK)], rowss[b], ssems[b]).wait()

    _eprefetch(0, s)
    _eprefetch(1, s + NSUB)
    _eprefetch(2, s + 2 * NSUB)
    _ewait(0)
    _fire(0)
    _ewait(1)
    _fire(1)

    def _piter(k, carry):
        for b in range(3):
            t = 3 * k + b
            _gdrain(b)
            _scatter(b)
            nb = (b + 2) % 3

            @pl.when(t + 2 < CPS)
            def _():
                @pl.when(t >= 1)
                def _():
                    _sdrain(nb)
                _ewait(nb)
                _fire(nb)

            @pl.when(t + 3 < CPS)
            def _():
                _eprefetch(b, s + (t + 3) * NSUB)
        return carry
    lax.fori_loop(0, CPS // 3, _piter, 0)
    _sdrain(0)
    _sdrain(1)
    _sdrain(2)
    plsc.subcore_barrier()

    iota2 = lax.iota(jnp.int32, 16) * 2
    nob = jnp.where(s < NOB % NSUB, NOB // NSUB + 1, NOB // NSUB)

    def _wblk(k, carry):
        b = s + k * NSUB
        off = b * OB
        pltpu.sync_copy(acc.at[pl.ds(off, OB)], stage.at[pl.ds(0, OB)])
        base = 2 * off + c
        for g in range(OB // 16):
            widx[0, pl.ds(g * 16, 16)] = base + 32 * g + iota2
        pltpu.sync_copy(stage.at[pl.ds(0, OB)], out.at[widx.at[0]])
        return carry
    lax.fori_loop(0, nob, _wblk, 0)

    @pl.when(s == NSUB - 1)
    def _():
        off = NOB * OB
        pltpu.sync_copy(acc.at[pl.ds(off, 32)], stage.at[pl.ds(0, 32)])
        base = 2 * off + c
        for g in range(2):
            widx[0, pl.ds(g * 16, 16)] = base + 32 * g + iota2
        pltpu.sync_copy(stage.at[pl.ds(0, 32)], out.at[widx.at[0, pl.ds(0, 32)]])


def _sc_segsum(h2, e4):
    return pl.kernel(
        _sc_segsum_body,
        out_type=jax.ShapeDtypeStruct((2 * N, HALF), jnp.float32),
        mesh=plsc.VectorSubcoreMesh(core_axis_name="c", subcore_axis_name="s"),
        compiler_params=pltpu.CompilerParams(use_tc_tiling_on_sc=False),
        scratch_types=[
            pltpu.VMEM((2, GPC, GRP), jnp.int32),
            pltpu.VMEM((2, GPC, GRP), jnp.int32),
            pltpu.VMEM((2, GPC, GRP), jnp.int32),
            pltpu.VMEM((GPC, GRP), jnp.int32),
            pltpu.VMEM((GPC, GRP), jnp.int32),
            pltpu.VMEM((GPC, GRP), jnp.int32),
            pltpu.VMEM((GPC, GRP), jnp.int32),
            pltpu.VMEM((GPC, GRP), jnp.int32),
            pltpu.VMEM((GPC, GRP), jnp.int32),
            pltpu.VMEM((CHUNK, HALF), jnp.float32),
            pltpu.VMEM((CHUNK, HALF), jnp.float32),
            pltpu.VMEM((CHUNK, HALF), jnp.float32),
            pltpu.VMEM((WB, HALF), jnp.float32),
            pltpu.VMEM((1, GRP), jnp.int32),
            pltpu.VMEM_SHARED((N + 8, HALF), jnp.float32),
            pltpu.SemaphoreType.DMA,
            pltpu.SemaphoreType.DMA,
            pltpu.SemaphoreType.DMA,
            pltpu.SemaphoreType.DMA,
            pltpu.SemaphoreType.DMA,
            pltpu.SemaphoreType.DMA,
            pltpu.SemaphoreType.DMA,
            pltpu.SemaphoreType.DMA,
            pltpu.SemaphoreType.DMA,
        ],
    )(h2, e4)


def _prep_body(x4_ref, w4_ref, out_ref):
    out_ref[...] = jnp.dot(x4_ref[0], w4_ref[...],
                           preferred_element_type=jnp.float32)


def _prep(x4, w4):
    return pl.pallas_call(
        _prep_body,
        grid=(GRID,),
        in_specs=[
            pl.BlockSpec((1, NB, 4), lambda i: (i, 0, 0)),
            pl.BlockSpec((4, 128), lambda i: (0, 0)),
        ],
        out_specs=pl.BlockSpec((NB, 128), lambda i: (i, 0)),
        out_shape=jax.ShapeDtypeStruct((NROW, 128), jnp.float32),
    )(x4, w4)


def _tc_mlp_body(h_ref, agg_ref, w1_ref, bias1_ref, w2_ref, bias2_ref,
                 hout_ref):
    z = h_ref[...] + agg_ref[...]
    z = jnp.maximum(
        jnp.dot(z, w1_ref[...], preferred_element_type=jnp.float32)
        + bias1_ref[...], 0.0)
    z = jnp.dot(z, w2_ref[...], preferred_element_type=jnp.float32) + bias2_ref[...]
    hout_ref[...] = jnp.maximum(z, 0.0)


def _tc_mlp(h128, agg128, w1d, b1d, w2d, b2d):
    return pl.pallas_call(
        _tc_mlp_body,
        grid=(GRID,),
        in_specs=[
            pl.BlockSpec((NB, 128), lambda i: (i, 0)),
            pl.BlockSpec((NB, 128), lambda i: (i, 0)),
            pl.BlockSpec((128, 128), lambda i: (0, 0)),
            pl.BlockSpec((1, 128), lambda i: (0, 0)),
            pl.BlockSpec((128, 128), lambda i: (0, 0)),
            pl.BlockSpec((1, 128), lambda i: (0, 0)),
        ],
        out_specs=pl.BlockSpec((NB, 128), lambda i: (i, 0)),
        out_shape=jax.ShapeDtypeStruct((NROW, 128), jnp.float32),
    )(h128, agg128, w1d, b1d, w2d, b2d)


def _tc_pool_body(h_ref, b4_ref, pool_ref):
    hn = h_ref[...]
    b4v = b4_ref[0]
    gids = lax.broadcasted_iota(jnp.int32, (NB, NG), 1)
    pool = jnp.zeros((NG, D), jnp.float32)
    for i in range(4):
        onehot = (b4v[:, i:i + 1] == gids).astype(jnp.float32)
        zi = hn[:, 32 * i:32 * (i + 1)]
        pool = pool + lax.dot_general(onehot, zi, (((0,), (0,)), ((), ())),
                                      preferred_element_type=jnp.float32)

    @pl.when(pl.program_id(0) == 0)
    def _():
        pool_ref[...] = jnp.zeros_like(pool_ref)

    pool_ref[...] += pool


def _tc_pool(h128, b4):
    return pl.pallas_call(
        _tc_pool_body,
        grid=(GRID,),
        in_specs=[
            pl.BlockSpec((NB, 128), lambda i: (i, 0)),
            pl.BlockSpec((1, NB, 4), lambda i: (i, 0, 0)),
        ],
        out_specs=pl.BlockSpec((NG, D), lambda i: (0, 0)),
        out_shape=jax.ShapeDtypeStruct((NG, D), jnp.float32),
    )(h128, b4)


def _head_body(g_ref, w1_ref, b1_ref, w2_ref, b2_ref, out_ref):
    g = jnp.maximum(
        jnp.dot(g_ref[...], w1_ref[...], preferred_element_type=jnp.float32)
        + b1_ref[...], 0.0)
    logits = jnp.dot(g, w2_ref[...], preferred_element_type=jnp.float32) + b2_ref[...]
    col = lax.broadcasted_iota(jnp.int32, (NG, 128), 1)
    valid = col < NCLS
    masked = jnp.where(valid, logits, -1e30)
    m = jnp.max(masked, axis=1, keepdims=True)
    e = jnp.where(valid, jnp.exp(logits - m), 0.0)
    lse = m + jnp.log(jnp.sum(e, axis=1, keepdims=True))
    out_ref[...] = logits - lse


def _head(g, w1, b1, w2, b2):
    return pl.pallas_call(
        _head_body,
        in_specs=[
            pl.BlockSpec((NG, 2 * 128), lambda: (0, 0)),
            pl.BlockSpec((2 * 128, D), lambda: (0, 0)),
            pl.BlockSpec((1, D), lambda: (0, 0)),
            pl.BlockSpec((D, 128), lambda: (0, 0)),
            pl.BlockSpec((1, 128), lambda: (0, 0)),
        ],
        out_specs=pl.BlockSpec((NG, 128), lambda: (0, 0)),
        out_shape=jax.ShapeDtypeStruct((NG, 128), jnp.float32),
    )(g, w1, b1, w2, b2)


def kernel(x, edge_index, batch,
           conv0_W1, conv0_b1, conv0_W2, conv0_b2,
           conv1_W1, conv1_b1, conv1_W2, conv1_b2,
           conv2_W1, conv2_b1, conv2_W2, conv2_b2,
           conv3_W1, conv3_b1, conv3_W2, conv3_b2,
           conv4_W1, conv4_b1, conv4_W2, conv4_b2,
           fc1_W, fc1_b, fc2_W, fc2_b):
    src_pad = jnp.concatenate(
        [edge_index[0], jnp.zeros((EPAD,), jnp.int32)])
    dst_pad = jnp.concatenate(
        [edge_index[1], N + (jnp.arange(EPAD, dtype=jnp.int32) % 8)])
    e4 = jnp.stack([src_pad.reshape(NCHUNK, GPC, GRP),
                    dst_pad.reshape(NCHUNK, GPC, GRP)], axis=1)

    b4 = batch.reshape(GRID, NB, 4)

    eye = jnp.eye(D, dtype=jnp.float32)
    i4 = jnp.eye(4, dtype=jnp.float32)
    w1s = [eye, conv1_W1, conv2_W1, conv3_W1, conv4_W1]
    b1s = [conv0_b1, conv1_b1, conv2_b1, conv3_b1, conv4_b1]
    w2s = [conv0_W2, conv1_W2, conv2_W2, conv3_W2, conv4_W2]
    b2s = [conv0_b2, conv1_b2, conv2_b2, conv3_b2, conv4_b2]

    w1d = [jnp.kron(i4, w) for w in w1s]
    w2d = [jnp.kron(i4, w) for w in w2s]
    b1d = [jnp.tile(b, 4).reshape(1, 128) for b in b1s]
    b2d = [jnp.tile(b, 4).reshape(1, 128) for b in b2s]

    x4 = x.reshape(GRID, NB, 4)
    w4 = jnp.kron(i4, conv0_W1)
    h128 = _prep(x4, w4)

    pooled = []
    for i in range(5):
        h2 = h128.reshape(2 * N, HALF)
        agg2 = _sc_segsum(h2, e4)
        agg128 = agg2.reshape(NROW, 128)
        h128 = _tc_mlp(h128, agg128, w1d[i], b1d[i], w2d[i], b2d[i])
        pooled.append(_tc_pool(h128, b4))

    g = jnp.concatenate(pooled, axis=1)
    gp = jnp.pad(g, ((0, 0), (0, 2 * 128 - 5 * D)))
    fc1p = jnp.pad(fc1_W, ((0, 2 * 128 - 5 * D), (0, 0)))
    fc2p = jnp.pad(fc2_W, ((0, 0), (0, 128 - NCLS)))
    fc2bp = jnp.pad(fc2_b, (0, 128 - NCLS))
    out = _head(gp, fc1p, fc1_b.reshape(1, D), fc2p, fc2bp.reshape(1, 128))
    return out[:, :NCLS]


# speedup vs baseline: 28.7852x; 1.0432x over previous
import jax
import jax.numpy as jnp
from jax import lax
from jax.experimental import pallas as pl
from jax.experimental.pallas import tpu as pltpu
from jax.experimental.pallas import tpu_sc as plsc

N = 100000
E = 1600000
D = 32
HALF = 16
NG = 128
NCLS = 6
NSUB = 16
NCORE = 2

GRP = 128
GPC = 3
CHUNK = GRP * GPC
NCHUNK = 4176
EPAD = NCHUNK * CHUNK - E
CPS = NCHUNK // NSUB
WB = 250
NWB = N // WB
OB = 128
NOB = N // OB

NROW = N // 4
NB = 5000
GRID = NROW // NB


def _sc_segsum_body(h2, e4, out,
                    ebuf0, ebuf1, ebuf2, idxv0, idxv1, idxv2,
                    dstv0, dstv1, dstv2,
                    rows0, rows1, rows2, stage, widx, acc,
                    g0, g1, g2, sc0, sc1, sc2, e0, e1, e2):
    c = lax.axis_index("c")
    s = lax.axis_index("s")
    ebufs = (ebuf0, ebuf1, ebuf2)
    idxvs = (idxv0, idxv1, idxv2)
    dstvs = (dstv0, dstv1, dstv2)
    rowss = (rows0, rows1, rows2)
    gsems = (g0, g1, g2)
    ssems = (sc0, sc1, sc2)
    esems = (e0, e1, e2)

    def _zrow(i, carry):
        stage[i] = jnp.zeros((HALF,), jnp.float32)
        return carry
    lax.fori_loop(0, WB, _zrow, 0)

    def _zblk(k, carry):
        b = s + k * NSUB
        pltpu.sync_copy(stage, acc.at[pl.ds(b * WB, WB)])
        return carry
    lax.fori_loop(0, NWB // NSUB, _zblk, 0)

    @pl.when(s == 0)
    def _():
        pltpu.sync_copy(stage.at[pl.ds(0, 8)], acc.at[pl.ds(N, 8)])
    plsc.subcore_barrier()

    def _eprefetch(b, ci):
        pltpu.async_copy(e4.at[ci], ebufs[b], esems[b])

    def _ewait(b):
        pltpu.make_async_copy(e4.at[0], ebufs[b], esems[b]).wait()

    def _fire(b):
        for j in range(GPC):
            for k in range(GRP // 16):
                v = ebufs[b][0, j, pl.ds(k * 16, 16)]
                idxvs[b][j, pl.ds(k * 16, 16)] = v * 2 + c
                dstvs[b][j, pl.ds(k * 16, 16)] = ebufs[b][1, j, pl.ds(k * 16, 16)]
        for j in range(GPC):
            pltpu.async_copy(h2.at[idxvs[b].at[j]],
                             rowss[b].at[pl.ds(j * GRP, GRP)], gsems[b])

    def _gdrain(b):
        pltpu.make_async_copy(out.at[pl.ds(0, CHUNK)], rowss[b], gsems[b]).wait()

    def _scatter(b):
        for j in range(GPC):
            pltpu.async_copy(rowss[b].at[pl.ds(j * GRP, GRP)],
                             acc.at[dstvs[b].at[j]], ssems[b], add=True)

    def _sdrain(b):
        pltpu.make_async_copy(out.at[pl.ds(0, CHUNK)], rowss[b], ssems[b]).wait()

    _eprefetch(0, s)
    _eprefetch(1, s + NSUB)
    _eprefetch(2, s + 2 * NSUB)
    _ewait(0)
    _fire(0)
    _ewait(1)
    _fire(1)

    def _piter(k, carry):
        for b in range(3):
            t = 3 * k + b
            _gdrain(b)
            _scatter(b)
            nb = (b + 2) % 3

            @pl.when(t + 2 < CPS)
            def _():
                @pl.when(t >= 1)
                def _():
                    _sdrain(nb)
                _ewait(nb)
                _fire(nb)

            @pl.when(t + 3 < CPS)
            def _():
                _eprefetch(b, s + (t + 3) * NSUB)
        return carry
    lax.fori_loop(0, CPS // 3, _piter, 0)
    _sdrain(0)
    _sdrain(1)
    _sdrain(2)
    plsc.subcore_barrier()

    iota2 = lax.iota(jnp.int32, 16) * 2
    nob = jnp.where(s < NOB % NSUB, NOB // NSUB + 1, NOB // NSUB)

    def _wblk(k, carry):
        b = s + k * NSUB
        off = b * OB
        pltpu.sync_copy(acc.at[pl.ds(off, OB)], stage.at[pl.ds(0, OB)])
        base = 2 * off + c
        for g in range(OB // 16):
            widx[0, pl.ds(g * 16, 16)] = base + 32 * g + iota2
        pltpu.sync_copy(stage.at[pl.ds(0, OB)], out.at[widx.at[0]])
        return carry
    lax.fori_loop(0, nob, _wblk, 0)

    @pl.when(s == NSUB - 1)
    def _():
        off = NOB * OB
        pltpu.sync_copy(acc.at[pl.ds(off, 32)], stage.at[pl.ds(0, 32)])
        base = 2 * off + c
        for g in range(2):
            widx[0, pl.ds(g * 16, 16)] = base + 32 * g + iota2
        pltpu.sync_copy(stage.at[pl.ds(0, 32)], out.at[widx.at[0, pl.ds(0, 32)]])


def _sc_segsum(h2, e4):
    return pl.kernel(
        _sc_segsum_body,
        out_type=jax.ShapeDtypeStruct((2 * N, HALF), jnp.float32),
        mesh=plsc.VectorSubcoreMesh(core_axis_name="c", subcore_axis_name="s"),
        compiler_params=pltpu.CompilerParams(use_tc_tiling_on_sc=False),
        scratch_types=[
            pltpu.VMEM((2, GPC, GRP), jnp.int32),
            pltpu.VMEM((2, GPC, GRP), jnp.int32),
            pltpu.VMEM((2, GPC, GRP), jnp.int32),
            pltpu.VMEM((GPC, GRP), jnp.int32),
            pltpu.VMEM((GPC, GRP), jnp.int32),
            pltpu.VMEM((GPC, GRP), jnp.int32),
            pltpu.VMEM((GPC, GRP), jnp.int32),
            pltpu.VMEM((GPC, GRP), jnp.int32),
            pltpu.VMEM((GPC, GRP), jnp.int32),
            pltpu.VMEM((CHUNK, HALF), jnp.float32),
            pltpu.VMEM((CHUNK, HALF), jnp.float32),
            pltpu.VMEM((CHUNK, HALF), jnp.float32),
            pltpu.VMEM((WB, HALF), jnp.float32),
            pltpu.VMEM((1, GRP), jnp.int32),
            pltpu.VMEM_SHARED((N + 8, HALF), jnp.float32),
            pltpu.SemaphoreType.DMA,
            pltpu.SemaphoreType.DMA,
            pltpu.SemaphoreType.DMA,
            pltpu.SemaphoreType.DMA,
            pltpu.SemaphoreType.DMA,
            pltpu.SemaphoreType.DMA,
            pltpu.SemaphoreType.DMA,
            pltpu.SemaphoreType.DMA,
            pltpu.SemaphoreType.DMA,
        ],
    )(h2, e4)


def _prep_body(x4_ref, w4_ref, out_ref):
    out_ref[...] = jnp.dot(x4_ref[0], w4_ref[...],
                           preferred_element_type=jnp.float32)


def _prep(x4, w4):
    return pl.pallas_call(
        _prep_body,
        grid=(GRID,),
        in_specs=[
            pl.BlockSpec((1, NB, 4), lambda i: (i, 0, 0)),
            pl.BlockSpec((4, 128), lambda i: (0, 0)),
        ],
        out_specs=pl.BlockSpec((NB, 128), lambda i: (i, 0)),
        out_shape=jax.ShapeDtypeStruct((NROW, 128), jnp.float32),
    )(x4, w4)


def _tc_mlp_body(h_ref, agg_ref, w1_ref, bias1_ref, w2_ref, bias2_ref,
                 hout_ref):
    z = h_ref[...] + agg_ref[...]
    z = jnp.maximum(
        jnp.dot(z, w1_ref[...], preferred_element_type=jnp.float32)
        + bias1_ref[...], 0.0)
    z = jnp.dot(z, w2_ref[...], preferred_element_type=jnp.float32) + bias2_ref[...]
    hout_ref[...] = jnp.maximum(z, 0.0)


def _tc_mlp(h128, agg128, w1d, b1d, w2d, b2d):
    return pl.pallas_call(
        _tc_mlp_body,
        grid=(GRID,),
        in_specs=[
            pl.BlockSpec((NB, 128), lambda i: (i, 0)),
            pl.BlockSpec((NB, 128), lambda i: (i, 0)),
            pl.BlockSpec((128, 128), lambda i: (0, 0)),
            pl.BlockSpec((1, 128), lambda i: (0, 0)),
            pl.BlockSpec((128, 128), lambda i: (0, 0)),
            pl.BlockSpec((1, 128), lambda i: (0, 0)),
        ],
        out_specs=pl.BlockSpec((NB, 128), lambda i: (i, 0)),
        out_shape=jax.ShapeDtypeStruct((NROW, 128), jnp.float32),
    )(h128, agg128, w1d, b1d, w2d, b2d)


def _tc_pool_body(h_ref, b4_ref, pool_ref):
    hn = h_ref[...]
    b4v = b4_ref[0]
    gids = lax.broadcasted_iota(jnp.int32, (NB, NG), 1)
    pool = jnp.zeros((NG, D), jnp.float32)
    for i in range(4):
        onehot = (b4v[:, i:i + 1] == gids).astype(jnp.float32)
        zi = hn[:, 32 * i:32 * (i + 1)]
        pool = pool + lax.dot_general(onehot, zi, (((0,), (0,)), ((), ())),
                                      preferred_element_type=jnp.float32)

    @pl.when(pl.program_id(0) == 0)
    def _():
        pool_ref[...] = jnp.zeros_like(pool_ref)

    pool_ref[...] += pool


def _tc_pool(h128, b4):
    return pl.pallas_call(
        _tc_pool_body,
        grid=(GRID,),
        in_specs=[
            pl.BlockSpec((NB, 128), lambda i: (i, 0)),
            pl.BlockSpec((1, NB, 4), lambda i: (i, 0, 0)),
        ],
        out_specs=pl.BlockSpec((NG, D), lambda i: (0, 0)),
        out_shape=jax.ShapeDtypeStruct((NG, D), jnp.float32),
    )(h128, b4)


def _head_body(g_ref, w1_ref, b1_ref, w2_ref, b2_ref, out_ref):
    g = jnp.maximum(
        jnp.dot(g_ref[...], w1_ref[...], preferred_element_type=jnp.float32)
        + b1_ref[...], 0.0)
    logits = jnp.dot(g, w2_ref[...], preferred_element_type=jnp.float32) + b2_ref[...]
    col = lax.broadcasted_iota(jnp.int32, (NG, 128), 1)
    valid = col < NCLS
    masked = jnp.where(valid, logits, -1e30)
    m = jnp.max(masked, axis=1, keepdims=True)
    e = jnp.where(valid, jnp.exp(logits - m), 0.0)
    lse = m + jnp.log(jnp.sum(e, axis=1, keepdims=True))
    out_ref[...] = logits - lse


def _head(g, w1, b1, w2, b2):
    return pl.pallas_call(
        _head_body,
        in_specs=[
            pl.BlockSpec((NG, 2 * 128), lambda: (0, 0)),
            pl.BlockSpec((2 * 128, D), lambda: (0, 0)),
            pl.BlockSpec((1, D), lambda: (0, 0)),
            pl.BlockSpec((D, 128), lambda: (0, 0)),
            pl.BlockSpec((1, 128), lambda: (0, 0)),
        ],
        out_specs=pl.BlockSpec((NG, 128), lambda: (0, 0)),
        out_shape=jax.ShapeDtypeStruct((NG, 128), jnp.float32),
    )(g, w1, b1, w2, b2)


def kernel(x, edge_index, batch,
           conv0_W1, conv0_b1, conv0_W2, conv0_b2,
           conv1_W1, conv1_b1, conv1_W2, conv1_b2,
           conv2_W1, conv2_b1, conv2_W2, conv2_b2,
           conv3_W1, conv3_b1, conv3_W2, conv3_b2,
           conv4_W1, conv4_b1, conv4_W2, conv4_b2,
           fc1_W, fc1_b, fc2_W, fc2_b):
    src_pad = jnp.concatenate(
        [edge_index[0], jnp.zeros((EPAD,), jnp.int32)])
    dst_pad = jnp.concatenate(
        [edge_index[1], N + (jnp.arange(EPAD, dtype=jnp.int32) % 8)])
    e4 = jnp.stack([src_pad.reshape(NCHUNK, GPC, GRP),
                    dst_pad.reshape(NCHUNK, GPC, GRP)], axis=1)

    b4 = batch.reshape(GRID, NB, 4)

    eye = jnp.eye(D, dtype=jnp.float32)
    i4 = jnp.eye(4, dtype=jnp.float32)
    w1s = [eye, conv1_W1, conv2_W1, conv3_W1, conv4_W1]
    b1s = [conv0_b1, conv1_b1, conv2_b1, conv3_b1, conv4_b1]
    w2s = [conv0_W2, conv1_W2, conv2_W2, conv3_W2, conv4_W2]
    b2s = [conv0_b2, conv1_b2, conv2_b2, conv3_b2, conv4_b2]

    w1d = [jnp.kron(i4, w) for w in w1s]
    w2d = [jnp.kron(i4, w) for w in w2s]
    b1d = [jnp.tile(b, 4).reshape(1, 128) for b in b1s]
    b2d = [jnp.tile(b, 4).reshape(1, 128) for b in b2s]

    x4 = x.reshape(GRID, NB, 4)
    w4 = jnp.kron(i4, conv0_W1)
    h128 = _prep(x4, w4)

    pooled = []
    for i in range(5):
        h2 = h128.reshape(2 * N, HALF)
        agg2 = _sc_segsum(h2, e4)
        agg128 = agg2.reshape(NROW, 128)
        h128 = _tc_mlp(h128, agg128, w1d[i], b1d[i], w2d[i], b2d[i])
        pooled.append(_tc_pool(h128, b4))

    g = jnp.concatenate(pooled, axis=1)
    gp = jnp.pad(g, ((0, 0), (0, 2 * 128 - 5 * D)))
    fc1p = jnp.pad(fc1_W, ((0, 2 * 128 - 5 * D), (0, 0)))
    fc2p = jnp.pad(fc2_W, ((0, 0), (0, 128 - NCLS)))
    fc2bp = jnp.pad(fc2_b, (0, 128 - NCLS))
    out = _head(gp, fc1p, fc1_b.reshape(1, D), fc2p, fc2bp.reshape(1, 128))
    return out[:, :NCLS]
